# baseline (speedup 1.0000x reference)
import functools

import jax
import jax.numpy as jnp
from jax import lax
from jax.experimental import pallas as pl
from jax.experimental.pallas import tpu as pltpu
from jax.experimental.pallas import tpu_sc as plsc

TABLE_ROWS = 100000
EMBED_D = 64
BATCH = 4096
SEQ = 26
NUM_CORES = 2
NUM_SUBCORES = 16
NW = NUM_CORES * NUM_SUBCORES
CHUNK = 128

_mesh = plsc.VectorSubcoreMesh(core_axis_name="c", subcore_axis_name="s")


@functools.partial(
    pl.kernel,
    mesh=_mesh,
    compiler_params=pltpu.CompilerParams(
        use_tc_tiling_on_sc=False, needs_layout_passes=False
    ),
    out_type=jax.ShapeDtypeStruct((SEQ, 8, NW, 8, CHUNK), jnp.float32),
    scratch_types=[
        pltpu.VMEM((SEQ, CHUNK), jnp.int32),
        pltpu.VMEM((2, CHUNK, EMBED_D), jnp.float32),
        pltpu.VMEM((2, 8, 8, CHUNK), jnp.float32),
        pltpu.SemaphoreType.DMA,
        pltpu.SemaphoreType.DMA,
        pltpu.SemaphoreType.DMA,
        pltpu.SemaphoreType.DMA,
    ],
)
def _gather_sc(idx_hbm, table_hbm, out_hbm, idx_v, rows_v, t_v, g0, g1, w0, w1):
    wid = lax.axis_index("s") * NUM_CORES + lax.axis_index("c")
    pltpu.sync_copy(idx_hbm.at[:, wid], idx_v)
    gs = (g0, g1)
    ws = (w0, w1)
    lanes = lax.broadcasted_iota(jnp.int32, (16,), 0)

    for b in range(2):
        pltpu.async_copy(table_hbm.at[idx_v.at[b]], rows_v.at[b], gs[b])

    def outer(j0, carry):
        for b in range(2):
            j = 2 * j0 + b

            @pl.when(j0 > 0)
            def _reclaim():
                pltpu.make_async_copy(
                    t_v.at[b], out_hbm.at[j - 2, :, wid], ws[b]
                ).wait()

            pltpu.make_async_copy(
                table_hbm.at[idx_v.at[j]], rows_v.at[b], gs[b]
            ).wait()

            rref = rows_v.at[b]
            tref = t_v.at[b]
            for cb in range(8):
                ridx = cb * 16 + lanes

                def dstep(tr, carry2, ridx=ridx, tref=tref, rref=rref):
                    for r in range(8):
                        d = tr * 8 + r
                        col = jnp.full((16,), d, jnp.int32)
                        x = plsc.load_gather(rref, [ridx, col])
                        tref[tr, r, pl.ds(cb * 16, 16)] = x
                    return carry2

                lax.fori_loop(0, 8, dstep, 0)

            pltpu.async_copy(tref, out_hbm.at[j, :, wid], ws[b])

            @pl.when(j < SEQ - 2)
            def _refill():
                pltpu.async_copy(
                    table_hbm.at[idx_v.at[j + 2]], rows_v.at[b], gs[b]
                )

        return carry

    lax.fori_loop(0, SEQ // 2, outer, 0)

    for b in range(2):
        pltpu.make_async_copy(
            t_v.at[b], out_hbm.at[SEQ - 2 + b, :, wid], ws[b]
        ).wait()


def kernel(inputs, embeddings):
    idx = inputs.astype(jnp.int32).T.reshape(SEQ, NW, CHUNK)
    out = _gather_sc(idx, embeddings)
    return out.transpose(2, 4, 0, 1, 3).reshape(BATCH, SEQ, EMBED_D)


# speedup vs baseline: 1.1477x; 1.1477x over previous
import functools

import jax
import jax.numpy as jnp
from jax import lax
from jax.experimental import pallas as pl
from jax.experimental.pallas import tpu as pltpu
from jax.experimental.pallas import tpu_sc as plsc

TABLE_ROWS = 100000
EMBED_D = 64
BATCH = 4096
SEQ = 26
NUM_CORES = 2
NUM_SUBCORES = 16
NW = NUM_CORES * NUM_SUBCORES
CHUNK = 128

_mesh = plsc.VectorSubcoreMesh(core_axis_name="c", subcore_axis_name="s")


@functools.partial(
    pl.kernel,
    mesh=_mesh,
    compiler_params=pltpu.CompilerParams(
        use_tc_tiling_on_sc=False, needs_layout_passes=False
    ),
    out_type=jax.ShapeDtypeStruct((SEQ, 8, NW, 8, CHUNK), jnp.float32),
    scratch_types=[
        pltpu.VMEM((SEQ, CHUNK), jnp.int32),
        pltpu.VMEM((2, CHUNK, EMBED_D), jnp.float32),
        pltpu.VMEM((2, 8, 8, CHUNK), jnp.float32),
        pltpu.SemaphoreType.DMA,
        pltpu.SemaphoreType.DMA,
        pltpu.SemaphoreType.DMA,
        pltpu.SemaphoreType.DMA,
    ],
)
def _gather_sc(idx_hbm, table_hbm, out_hbm, idx_v, rows_v, t_v, g0, g1, w0, w1):
    wid = lax.axis_index("s") * NUM_CORES + lax.axis_index("c")
    pltpu.sync_copy(idx_hbm.at[:, wid], idx_v)
    gs = (g0, g1)
    ws = (w0, w1)
    lanes = lax.broadcasted_iota(jnp.int32, (16,), 0)
    trs = [(d0 + lanes) >> 3 for d0 in (0, 16, 32, 48)]
    rrs = [(d0 + lanes) & 7 for d0 in (0, 16, 32, 48)]

    def transpose(rref, tref):
        def cstep(c0, carry):
            for ci in range(8):
                c = c0 * 8 + ci
                cvec = jnp.full((16,), c, jnp.int32)
                for k, d0 in enumerate((0, 16, 32, 48)):
                    x = rref[c, pl.ds(d0, 16)]
                    plsc.store_scatter(tref, [trs[k], rrs[k], cvec], x)
            return carry

        lax.fori_loop(0, 16, cstep, 0)

    for b in range(2):
        pltpu.async_copy(table_hbm.at[idx_v.at[b]], rows_v.at[b], gs[b])

    def outer(j0, carry):
        for b in range(2):
            j = 2 * j0 + b

            @pl.when(j0 > 0)
            def _reclaim():
                pltpu.make_async_copy(
                    t_v.at[b], out_hbm.at[j - 2, :, wid], ws[b]
                ).wait()

            pltpu.make_async_copy(
                table_hbm.at[idx_v.at[j]], rows_v.at[b], gs[b]
            ).wait()

            transpose(rows_v.at[b], t_v.at[b])

            pltpu.async_copy(t_v.at[b], out_hbm.at[j, :, wid], ws[b])

            @pl.when(j < SEQ - 2)
            def _refill():
                pltpu.async_copy(
                    table_hbm.at[idx_v.at[j + 2]], rows_v.at[b], gs[b]
                )

        return carry

    lax.fori_loop(0, SEQ // 2, outer, 0)

    for b in range(2):
        pltpu.make_async_copy(
            t_v.at[b], out_hbm.at[SEQ - 2 + b, :, wid], ws[b]
        ).wait()


def kernel(inputs, embeddings):
    idx = inputs.astype(jnp.int32).T.reshape(SEQ, NW, CHUNK)
    out = _gather_sc(idx, embeddings)
    return out.transpose(2, 4, 0, 1, 3).reshape(BATCH, SEQ, EMBED_D)
